# baseline (speedup 1.0000x reference)
import functools

import jax
import jax.numpy as jnp
from jax import lax
from jax.experimental import pallas as pl
from jax.experimental.pallas import tpu as pltpu
from jax.experimental.pallas import tpu_sc as plsc

NUM_CORES = 2
NUM_SUBCORES = 16
NW = NUM_CORES * NUM_SUBCORES
LANES = 16
DMA_GROUP = 16


def _sc_gather_pool(xi, catf, emb, ctab_p):
  B = xi.shape[0]
  D = emb.shape[1]
  n_cat = catf.shape[0] // B
  bpw = B // NW
  half = bpw // 2
  assert B % (8 * NW) == 0

  mesh = plsc.VectorSubcoreMesh(core_axis_name="c", subcore_axis_name="s")

  @functools.partial(
      pl.kernel,
      out_type=(
          jax.ShapeDtypeStruct((B, D), jnp.float32),
          jax.ShapeDtypeStruct((B * D,), jnp.float32),
      ),
      mesh=mesh,
      compiler_params=pltpu.CompilerParams(
          use_tc_tiling_on_sc=True, needs_layout_passes=False),
      scratch_types=[
          pltpu.VMEM((bpw,), jnp.int32),
          pltpu.VMEM((half,), jnp.int32),
          pltpu.VMEM((bpw * n_cat,), jnp.int32),
          pltpu.VMEM((half, 128), jnp.float32),
          pltpu.VMEM((bpw * D,), jnp.float32),
          pltpu.SemaphoreType.DMA,
          pltpu.SemaphoreType.DMA,
      ],
  )
  def k(xi_hbm, catf_hbm, emb_hbm, ctab_hbm, uout_hbm, cout_hbm,
        idx_v, cidx_v, catblk_v, tmp_v, acc_v, sem, sem2):
    wid = lax.axis_index("s") * NUM_CORES + lax.axis_index("c")
    base = wid * bpw
    iota = lax.iota(jnp.int32, LANES)
    iota_nc = iota * n_cat

    pltpu.sync_copy(xi_hbm.at[pl.ds(base, bpw)], idx_v)
    pltpu.sync_copy(catf_hbm.at[pl.ds(base * n_cat, bpw * n_cat)], catblk_v)

    def ugrp(g, _):
      vec = idx_v[pl.ds(g * LANES, LANES)]
      for t in range(LANES):
        pltpu.async_copy(emb_hbm.at[pl.ds(vec[t], 1)],
                         uout_hbm.at[pl.ds(base + g * LANES + t, 1)], sem)
      for _ in range(LANES):
        pltpu.make_async_copy(
            emb_hbm.at[pl.ds(0, 1)], uout_hbm.at[pl.ds(base, 1)], sem).wait()
      return 0

    lax.fori_loop(0, bpw // LANES, ugrp, 0)

    for h in range(2):
      hbase = h * half
      for j in range(n_cat):

        def ccol(c, _):
          flat = iota_nc + ((hbase + c * LANES) * n_cat + j)
          cidx_v[pl.ds(c * LANES, LANES)] = plsc.load_gather(
              catblk_v, [flat])
          return 0

        lax.fori_loop(0, half // LANES, ccol, 0)
        pltpu.async_copy(ctab_hbm.at[cidx_v], tmp_v, sem2).wait()

        if j == 0:
          def init(i, _):
            for cc in range(D // LANES):
              acc_v[pl.ds((hbase + i) * D + cc * LANES, LANES)] = (
                  tmp_v[i, pl.ds(cc * LANES, LANES)])
            return 0

          lax.fori_loop(0, half, init, 0)
        else:
          def accum(i, _):
            for cc in range(D // LANES):
              v = tmp_v[i, pl.ds(cc * LANES, LANES)]
              plsc.addupdate(
                  acc_v.at[pl.ds((hbase + i) * D + cc * LANES, LANES)], v)
            return 0

          lax.fori_loop(0, half, accum, 0)

    pltpu.sync_copy(acc_v, cout_hbm.at[pl.ds(base * D, bpw * D)])

  return k(xi, catf, emb, ctab_p)


def _tc_mlp(u, cp, numz, w1u, w1c, w1n, b1, w2, b2):
  B, D = u.shape
  H = w1u.shape[1]
  NP = numz.shape[1]
  BLK = 2048
  grid = (B // BLK,)

  def body(u_ref, c_ref, n_ref, w1u_ref, w1c_ref, w1n_ref, b1_ref, w2_ref,
           b2_ref, o_ref):
    h = jnp.dot(u_ref[...], w1u_ref[...], preferred_element_type=jnp.float32)
    h = h + jnp.dot(c_ref[...], w1c_ref[...],
                    preferred_element_type=jnp.float32)
    h = h + jnp.dot(n_ref[...], w1n_ref[...],
                    preferred_element_type=jnp.float32)
    h = jnp.maximum(h + b1_ref[...], 0.0)
    o_ref[...] = (jnp.dot(h, w2_ref[...], preferred_element_type=jnp.float32)
                  + b2_ref[0, 0])

  return pl.pallas_call(
      body,
      grid=grid,
      in_specs=[
          pl.BlockSpec((BLK, D), lambda i: (i, 0)),
          pl.BlockSpec((BLK, D), lambda i: (i, 0)),
          pl.BlockSpec((BLK, NP), lambda i: (i, 0)),
          pl.BlockSpec((D, H), lambda i: (0, 0)),
          pl.BlockSpec((D, H), lambda i: (0, 0)),
          pl.BlockSpec((NP, H), lambda i: (0, 0)),
          pl.BlockSpec((1, H), lambda i: (0, 0)),
          pl.BlockSpec((H, 1), lambda i: (0, 0)),
          pl.BlockSpec(memory_space=pltpu.SMEM),
      ],
      out_specs=pl.BlockSpec((BLK, 1), lambda i: (i, 0)),
      out_shape=jax.ShapeDtypeStruct((B, 1), jnp.float32),
  )(u, cp, numz, w1u, w1c, w1n, b1, w2, b2)


def kernel(x, category, numeric, emb, cat_table, W1, b1, W2, b2):
  B = x.shape[0]
  D = emb.shape[1]
  n_num = numeric.shape[1]

  xi = x[:, 0].astype(jnp.int32)
  catf = category.astype(jnp.int32).reshape(-1)
  ctab_p = jnp.pad(cat_table, ((0, 0), (0, 128 - D)))

  user_emb, cflat = _sc_gather_pool(xi, catf, emb, ctab_p)
  cat_pooled = cflat.reshape(B, D)

  np_pad = 16
  numz = jnp.pad(numeric, ((0, 0), (0, np_pad - n_num)))
  w1u = W1[:D]
  w1c = W1[D:2 * D]
  w1n = jnp.pad(W1[2 * D:], ((0, np_pad - n_num), (0, 0)))
  b1r = b1.reshape(1, -1)
  b2r = b2.reshape(1, 1)

  return _tc_mlp(user_emb, cat_pooled, numz, w1u, w1c, w1n, b1r, W2, b2r)
